# Initial kernel scaffold; baseline (speedup 1.0000x reference)
import jax
import jax.numpy as jnp
from jax.experimental import pallas as pl


def kernel(x, conv1_w, conv1_b, conv2_w, conv2_b, fc1_w, fc1_b, fc2_w, fc2_b, fc3_w, fc3_b, fc4_w, fc4_b):
    raise NotImplementedError("write your pallas kernel here")


# speedup vs baseline: 7.4808x; 7.4808x over previous
import numpy as np

import jax
import jax.numpy as jnp
from jax.experimental import pallas as pl
from jax.experimental.pallas import tpu as pltpu

_BC = 8
_K = 5


def _band(n_in, n_out):
    m = np.zeros((_K, n_in, n_out), np.float32)
    r = np.arange(n_out)
    for dj in range(_K):
        m[dj, r + dj, r] = 1.0
    return jnp.asarray(m)


def _sel_lanes(cout, wo, wp):
    m = np.zeros((cout * wo - 2, cout * wp), np.float32)
    for co in range(cout):
        for j in range(wp):
            m[co * wo + 3 * j, co * wp + j] = 1.0
    return jnp.asarray(m)


def _sel_rows(bc, stride, hp):
    m = np.zeros((bc * hp, bc * stride - 6), np.float32)
    for b in range(bc):
        for r in range(hp):
            m[b * hp + r, b * stride + 3 * r] = 1.0
    return jnp.asarray(m)


def _shift_sum(parts, rows):
    acc = parts[0][0:rows]
    for d in range(1, len(parts)):
        acc = acc + parts[d][d:d + rows]
    return acc


def _max3_lanes(a):
    return jnp.maximum(jnp.maximum(a[:, :-2], a[:, 1:-1]), a[:, 2:])


def _max3_rows(a):
    return jnp.maximum(jnp.maximum(a[:-2], a[1:-1]), a[2:])


def _conv_features(x, w1, b1, w2, b2):
    B, _, H, W = x.shape
    C1 = w1.shape[0]
    C2 = w2.shape[0]
    Ho1, Wo1 = H - _K + 1, W - _K + 1
    Hp1, Wp1 = Ho1 // 3, Wo1 // 3
    Ho2, Wo2 = Hp1 - _K + 1, Wp1 - _K + 1
    Hp2, Wp2 = Ho2 // 3, Wo2 // 3

    Bp = (B + _BC - 1) // _BC * _BC
    if Bp != B:
        x = jnp.pad(x, ((0, Bp - B), (0, 0), (0, 0), (0, 0)))
    xr = x.reshape(Bp * H, W)

    m1 = jnp.einsum("bjq,cdb->djcq", _band(W, Wo1), w1[:, 0])
    m1 = m1.reshape(_K, W, C1 * Wo1)
    m2 = jnp.einsum("bjq,cidb->dijcq", _band(Wp1, Wo2), w2)
    m2 = m2.reshape(_K, C1 * Wp1, C2 * Wo2)

    sw1 = _sel_lanes(C1, Wo1, Wp1)
    sh1 = _sel_rows(_BC, H, Hp1)
    sw2 = _sel_lanes(C2, Wo2, Wp2)
    sh2 = _sel_rows(_BC, Hp1, Hp2)

    b1row = jnp.repeat(b1, Wp1).reshape(1, C1 * Wp1)
    b2row = jnp.repeat(b2, Wp2).reshape(1, C2 * Wp2)

    R1 = _BC * H
    R1o = R1 - (_K - 1)
    R2 = _BC * Hp1
    R2o = R2 - (_K - 1)

    def _body(x_ref, m1_ref, m2_ref, sw1_ref, sh1_ref, sw2_ref, sh2_ref,
              b1_ref, b2_ref, o_ref):
        x1 = x_ref[...]
        p = [jnp.dot(x1, m1_ref[d], preferred_element_type=jnp.float32)
             for d in range(_K)]
        c1 = _shift_sum(p, R1o)
        pw = jnp.dot(_max3_lanes(c1), sw1_ref[...],
                     preferred_element_type=jnp.float32)
        a1 = jnp.dot(sh1_ref[...], _max3_rows(pw),
                     preferred_element_type=jnp.float32)
        x2 = jnp.maximum(a1 + b1_ref[...], 0.0)
        q = [jnp.dot(x2, m2_ref[d], preferred_element_type=jnp.float32)
             for d in range(_K)]
        c2 = _shift_sum(q, R2o)
        pw2 = jnp.dot(_max3_lanes(c2), sw2_ref[...],
                      preferred_element_type=jnp.float32)
        a2 = jnp.dot(sh2_ref[...], _max3_rows(pw2),
                     preferred_element_type=jnp.float32)
        o_ref[...] = jnp.maximum(a2 + b2_ref[...], 0.0)

    def _resident(a):
        return pl.BlockSpec(a.shape, lambda i: (0,) * a.ndim)

    feats = pl.pallas_call(
        _body,
        out_shape=jax.ShapeDtypeStruct((Bp * Hp2, C2 * Wp2), jnp.float32),
        grid=(Bp // _BC,),
        in_specs=[pl.BlockSpec((R1, W), lambda i: (i, 0)),
                  _resident(m1), _resident(m2),
                  _resident(sw1), _resident(sh1),
                  _resident(sw2), _resident(sh2),
                  _resident(b1row), _resident(b2row)],
        out_specs=pl.BlockSpec((_BC * Hp2, C2 * Wp2), lambda i: (i, 0)),
        compiler_params=pltpu.CompilerParams(
            dimension_semantics=("parallel",),
            vmem_limit_bytes=64 * 1024 * 1024),
    )(xr, m1, m2, sw1, sh1, sw2, sh2, b1row, b2row)
    return feats[:B * Hp2]


def _fc_chain(h0, w1, b1, w2, b2, w3, b3, w4, b4):
    B, K0 = h0.shape
    N = w4.shape[1]
    Bt = 512
    Bp = (B + Bt - 1) // Bt * Bt
    if Bp != B:
        h0 = jnp.pad(h0, ((0, Bp - B), (0, 0)))
    rows = [v.reshape(1, -1) for v in (b1, b2, b3, b4)]

    def _body(x_ref, w1r, b1r, w2r, b2r, w3r, b3r, w4r, b4r, o_ref):
        h = x_ref[...]
        for wr, br, last in ((w1r, b1r, False), (w2r, b2r, False),
                             (w3r, b3r, False), (w4r, b4r, True)):
            h = jnp.dot(h, wr[...], preferred_element_type=jnp.float32) + br[...]
            if not last:
                h = jnp.maximum(h, 0.0)
        o_ref[...] = h

    def _resident(a):
        return pl.BlockSpec(a.shape, lambda i: (0, 0))

    out = pl.pallas_call(
        _body,
        out_shape=jax.ShapeDtypeStruct((Bp, N), jnp.float32),
        grid=(Bp // Bt,),
        in_specs=[pl.BlockSpec((Bt, K0), lambda i: (i, 0)),
                  _resident(w1), _resident(rows[0]),
                  _resident(w2), _resident(rows[1]),
                  _resident(w3), _resident(rows[2]),
                  _resident(w4), _resident(rows[3])],
        out_specs=pl.BlockSpec((Bt, N), lambda i: (i, 0)),
        compiler_params=pltpu.CompilerParams(
            dimension_semantics=("parallel",)),
    )(h0, w1, rows[0], w2, rows[1], w3, rows[2], w4, rows[3])
    return out[:B]


def kernel(x, conv1_w, conv1_b, conv2_w, conv2_b,
           fc1_w, fc1_b, fc2_w, fc2_b, fc3_w, fc3_b, fc4_w, fc4_b):
    x = x.astype(jnp.float32)
    B = x.shape[0]
    C2 = conv2_w.shape[0]
    H, W = x.shape[2], x.shape[3]
    Hp2 = ((H - _K + 1) // 3 - _K + 1) // 3
    Wp2 = ((W - _K + 1) // 3 - _K + 1) // 3

    feats = _conv_features(x, conv1_w, conv1_b, conv2_w, conv2_b)
    flat = feats.reshape(B, Hp2 * C2 * Wp2)
    fc1_wp = fc1_w.reshape(C2, Hp2, Wp2, -1).transpose(1, 0, 2, 3)
    fc1_wp = fc1_wp.reshape(Hp2 * C2 * Wp2, -1)
    return _fc_chain(flat, fc1_wp, fc1_b, fc2_w, fc2_b,
                     fc3_w, fc3_b, fc4_w, fc4_b)
